# baseline (speedup 1.0000x reference)
import functools

import jax
import jax.numpy as jnp
from jax import lax
from jax.experimental import pallas as pl
from jax.experimental.pallas import tpu as pltpu
from jax.experimental.pallas import tpu_sc as plsc

IN_F = 512
OUT_F = 32
LANES = 16


def _build_mask_sc(idx, B, Q, M):
    mesh = plsc.VectorSubcoreMesh(core_axis_name="c", subcore_axis_name="s")

    @functools.partial(
        pl.kernel,
        mesh=mesh,
        out_type=jax.ShapeDtypeStruct((B, Q), jnp.float32),
        scratch_types=[
            pltpu.VMEM((M,), jnp.int32),
            pltpu.VMEM((Q,), jnp.float32),
        ],
        compiler_params=pltpu.CompilerParams(needs_layout_passes=False),
    )
    def mask_kernel(idx_hbm, mask_hbm, idx_v, mask_v):
        num_c = lax.axis_size("c")
        wid = lax.axis_index("s") * num_c + lax.axis_index("c")

        @pl.when(wid < B)
        def _():
            pltpu.sync_copy(idx_hbm.at[wid], idx_v)

            zeros = jnp.zeros((LANES,), jnp.float32)

            def zero_body(i, carry):
                mask_v[pl.ds(i * LANES, LANES)] = zeros
                return carry

            lax.fori_loop(0, Q // LANES, zero_body, 0)

            ones = jnp.ones((LANES,), jnp.float32)

            def scat_body(i, carry):
                ids = idx_v[pl.ds(i * LANES, LANES)]
                plsc.store_scatter(mask_v, [ids], ones)
                return carry

            lax.fori_loop(0, M // LANES, scat_body, 0)

            pltpu.sync_copy(mask_v, mask_hbm.at[wid])

    return mask_kernel(idx)


def _dense_linear_tc(d_A, WT, b2, B, Q):
    grid = (B,)

    def body(x_ref, wt_ref, b_ref, o_ref):
        acc = jnp.dot(x_ref[0], wt_ref[...], preferred_element_type=jnp.float32)
        o_ref[0] = acc + b_ref[...]

    return pl.pallas_call(
        body,
        grid=grid,
        in_specs=[
            pl.BlockSpec((1, Q, IN_F), lambda b: (b, 0, 0)),
            pl.BlockSpec((IN_F, OUT_F), lambda b: (0, 0)),
            pl.BlockSpec((1, OUT_F), lambda b: (0, 0)),
        ],
        out_specs=pl.BlockSpec((1, Q, OUT_F), lambda b: (b, 0, 0)),
        out_shape=jax.ShapeDtypeStruct((B, Q, OUT_F), d_A.dtype),
    )(d_A, WT, b2)


def _apply_mask_tc(up, mask3, B, Q):
    grid = (B,)

    def body(u_ref, m_ref, o_ref):
        ones_row = jnp.ones((1, OUT_F), jnp.float32)
        mcol = lax.dot_general(
            m_ref[0],
            ones_row,
            (((0,), (0,)), ((), ())),
            preferred_element_type=jnp.float32,
        )
        o_ref[0] = u_ref[0] * mcol

    return pl.pallas_call(
        body,
        grid=grid,
        in_specs=[
            pl.BlockSpec((1, Q, OUT_F), lambda b: (b, 0, 0)),
            pl.BlockSpec((1, 1, Q), lambda b: (b, 0, 0)),
        ],
        out_specs=pl.BlockSpec((1, Q, OUT_F), lambda b: (b, 0, 0)),
        out_shape=jax.ShapeDtypeStruct((B, Q, OUT_F), up.dtype),
    )(up, mask3)


def kernel(d_A, masked_indices_list, W, b):
    B, Q, _ = d_A.shape
    M = masked_indices_list.shape[1]
    idx = masked_indices_list.astype(jnp.int32)
    mask = _build_mask_sc(idx, B, Q, M)
    mask3 = mask.reshape(B, 1, Q)
    up = _dense_linear_tc(d_A, W.T, b.reshape(1, OUT_F), B, Q)
    return _apply_mask_tc(up, mask3, B, Q)


# speedup vs baseline: 1.2440x; 1.2440x over previous
import functools

import jax
import jax.numpy as jnp
from jax import lax
from jax.experimental import pallas as pl
from jax.experimental.pallas import tpu as pltpu
from jax.experimental.pallas import tpu_sc as plsc

IN_F = 512
OUT_F = 32
LANES = 16


def _build_mask_sc(idx, B, Q, M):
    mesh = plsc.VectorSubcoreMesh(core_axis_name="c", subcore_axis_name="s")

    @functools.partial(
        pl.kernel,
        mesh=mesh,
        out_type=jax.ShapeDtypeStruct((B, Q), jnp.float32),
        scratch_types=[
            pltpu.VMEM((M,), jnp.int32),
            pltpu.VMEM((Q,), jnp.float32),
        ],
        compiler_params=pltpu.CompilerParams(needs_layout_passes=False),
    )
    def mask_kernel(idx_hbm, mask_hbm, idx_v, mask_v):
        num_c = lax.axis_size("c")
        wid = lax.axis_index("s") * num_c + lax.axis_index("c")

        @pl.when(wid < B)
        def _():
            pltpu.sync_copy(idx_hbm.at[wid], idx_v)

            zeros = jnp.zeros((LANES,), jnp.float32)

            def zero_body(i, carry):
                mask_v[pl.ds(i * LANES, LANES)] = zeros
                return carry

            lax.fori_loop(0, Q // LANES, zero_body, 0)

            ones = jnp.ones((LANES,), jnp.float32)

            def scat_body(i, carry):
                ids = idx_v[pl.ds(i * LANES, LANES)]
                plsc.store_scatter(mask_v, [ids], ones)
                return carry

            lax.fori_loop(0, M // LANES, scat_body, 0)

            pltpu.sync_copy(mask_v, mask_hbm.at[wid])

    return mask_kernel(idx)


def _masked_linear_tc(d_A, mask, WT, b2, B, Q):
    grid = (B,)

    def body(x_ref, m_ref, wt_ref, b_ref, o_ref):
        acc = jnp.dot(x_ref[0], wt_ref[...], preferred_element_type=jnp.float32)
        m_row = m_ref[pl.ds(pl.program_id(0), 1), :]
        ones_row = jnp.ones((1, OUT_F), jnp.float32)
        mcol = lax.dot_general(
            m_row,
            ones_row,
            (((0,), (0,)), ((), ())),
            preferred_element_type=jnp.float32,
        )
        o_ref[0] = (acc + b_ref[...]) * mcol

    return pl.pallas_call(
        body,
        grid=grid,
        in_specs=[
            pl.BlockSpec((1, Q, IN_F), lambda b: (b, 0, 0)),
            pl.BlockSpec((B, Q), lambda b: (0, 0)),
            pl.BlockSpec((IN_F, OUT_F), lambda b: (0, 0)),
            pl.BlockSpec((1, OUT_F), lambda b: (0, 0)),
        ],
        out_specs=pl.BlockSpec((1, Q, OUT_F), lambda b: (b, 0, 0)),
        out_shape=jax.ShapeDtypeStruct((B, Q, OUT_F), d_A.dtype),
    )(d_A, mask, WT, b2)


def kernel(d_A, masked_indices_list, W, b):
    B, Q, _ = d_A.shape
    M = masked_indices_list.shape[1]
    idx = masked_indices_list.astype(jnp.int32)
    mask = _build_mask_sc(idx, B, Q, M)
    return _masked_linear_tc(d_A, mask, W.T, b.reshape(1, OUT_F), B, Q)


# speedup vs baseline: 1.2521x; 1.0065x over previous
import functools

import jax
import jax.numpy as jnp
from jax import lax
from jax.experimental import pallas as pl
from jax.experimental.pallas import tpu as pltpu
from jax.experimental.pallas import tpu_sc as plsc

IN_F = 512
OUT_F = 32
LANES = 16


def _build_mask_sc(idx, B, Q, M):
    half = Q // 2
    mesh = plsc.VectorSubcoreMesh(core_axis_name="c", subcore_axis_name="s")

    @functools.partial(
        pl.kernel,
        mesh=mesh,
        out_type=jax.ShapeDtypeStruct((B, Q), jnp.float32),
        scratch_types=[
            pltpu.VMEM((M,), jnp.int32),
            pltpu.VMEM((half,), jnp.float32),
        ],
        compiler_params=pltpu.CompilerParams(needs_layout_passes=False),
    )
    def mask_kernel(idx_hbm, mask_hbm, idx_v, mask_v):
        num_c = lax.axis_size("c")
        wid = lax.axis_index("s") * num_c + lax.axis_index("c")
        batch = wid // 2
        base = (wid % 2) * half

        pltpu.sync_copy(idx_hbm.at[batch], idx_v)

        zeros = jnp.zeros((LANES,), jnp.float32)

        def zero_body(i, carry):
            mask_v[pl.ds(i * LANES, LANES)] = zeros
            return carry

        lax.fori_loop(0, half // LANES, zero_body, 0)

        ones = jnp.ones((LANES,), jnp.float32)

        def scat_body(i, carry):
            ids = idx_v[pl.ds(i * LANES, LANES)]
            local = ids - base
            ok = (local >= 0) & (local < half)
            safe = jnp.where(ok, local, 0)
            plsc.store_scatter(mask_v, [safe], ones, mask=ok)
            return carry

        lax.fori_loop(0, M // LANES, scat_body, 0)

        pltpu.sync_copy(mask_v, mask_hbm.at[batch, pl.ds(base, half)])

    return mask_kernel(idx)


def _masked_linear_tc(d_A, mask, WT, b2, B, Q):
    grid = (B,)

    def body(x_ref, m_ref, wt_ref, b_ref, o_ref):
        acc = jnp.dot(x_ref[0], wt_ref[...], preferred_element_type=jnp.float32)
        m_row = m_ref[pl.ds(pl.program_id(0), 1), :]
        ones_row = jnp.ones((1, OUT_F), jnp.float32)
        mcol = lax.dot_general(
            m_row,
            ones_row,
            (((0,), (0,)), ((), ())),
            preferred_element_type=jnp.float32,
        )
        o_ref[0] = (acc + b_ref[...]) * mcol

    return pl.pallas_call(
        body,
        grid=grid,
        in_specs=[
            pl.BlockSpec((1, Q, IN_F), lambda b: (b, 0, 0)),
            pl.BlockSpec((B, Q), lambda b: (0, 0)),
            pl.BlockSpec((IN_F, OUT_F), lambda b: (0, 0)),
            pl.BlockSpec((1, OUT_F), lambda b: (0, 0)),
        ],
        out_specs=pl.BlockSpec((1, Q, OUT_F), lambda b: (b, 0, 0)),
        out_shape=jax.ShapeDtypeStruct((B, Q, OUT_F), d_A.dtype),
    )(d_A, mask, WT, b2)


def kernel(d_A, masked_indices_list, W, b):
    B, Q, _ = d_A.shape
    M = masked_indices_list.shape[1]
    idx = masked_indices_list.astype(jnp.int32)
    mask = _build_mask_sc(idx, B, Q, M)
    return _masked_linear_tc(d_A, mask, W.T, b.reshape(1, OUT_F), B, Q)


# speedup vs baseline: 1.2533x; 1.0009x over previous
import functools

import jax
import jax.numpy as jnp
from jax import lax
from jax.experimental import pallas as pl
from jax.experimental.pallas import tpu as pltpu
from jax.experimental.pallas import tpu_sc as plsc

IN_F = 512
OUT_F = 32
LANES = 16


def _build_mask_sc(idx, B, Q, M):
    half = Q // 2
    mesh = plsc.VectorSubcoreMesh(core_axis_name="c", subcore_axis_name="s")

    @functools.partial(
        pl.kernel,
        mesh=mesh,
        out_type=jax.ShapeDtypeStruct((B, Q), jnp.float32),
        scratch_types=[
            pltpu.VMEM((M,), jnp.int32),
            pltpu.VMEM((half,), jnp.float32),
        ],
        compiler_params=pltpu.CompilerParams(needs_layout_passes=False),
    )
    def mask_kernel(idx_hbm, mask_hbm, idx_v, mask_v):
        num_c = lax.axis_size("c")
        wid = lax.axis_index("s") * num_c + lax.axis_index("c")
        batch = wid // 2
        base = (wid % 2) * half

        pltpu.sync_copy(idx_hbm.at[batch], idx_v)

        zeros = jnp.zeros((LANES,), jnp.float32)

        def zero_body(i, carry):
            mask_v[pl.ds(i * LANES, LANES)] = zeros
            return carry

        lax.fori_loop(0, half // LANES, zero_body, 0)

        ones = jnp.ones((LANES,), jnp.float32)

        def scat_body(i, carry):
            ids = idx_v[pl.ds(i * LANES, LANES)]
            local = ids - base
            ok = (local >= 0) & (local < half)
            safe = jnp.where(ok, local, 0)
            plsc.store_scatter(mask_v, [safe], ones, mask=ok)
            return carry

        lax.fori_loop(0, M // LANES, scat_body, 0)

        pltpu.sync_copy(mask_v, mask_hbm.at[batch, pl.ds(base, half)])

    return mask_kernel(idx)


def _masked_linear_tc(d_A, mask, W, b2, B, Q):
    grid = (B,)

    def body(x_ref, m_ref, w_ref, b_ref, o_ref):
        acc = lax.dot_general(
            x_ref[0],
            w_ref[...],
            (((1,), (1,)), ((), ())),
            preferred_element_type=jnp.float32,
        )
        m_row = m_ref[pl.ds(pl.program_id(0), 1), :]
        ones_row = jnp.ones((1, OUT_F), jnp.float32)
        mcol = lax.dot_general(
            m_row,
            ones_row,
            (((0,), (0,)), ((), ())),
            preferred_element_type=jnp.float32,
        )
        o_ref[0] = (acc + b_ref[...]) * mcol

    return pl.pallas_call(
        body,
        grid=grid,
        in_specs=[
            pl.BlockSpec((1, Q, IN_F), lambda b: (b, 0, 0)),
            pl.BlockSpec((B, Q), lambda b: (0, 0)),
            pl.BlockSpec((OUT_F, IN_F), lambda b: (0, 0)),
            pl.BlockSpec((1, OUT_F), lambda b: (0, 0)),
        ],
        out_specs=pl.BlockSpec((1, Q, OUT_F), lambda b: (b, 0, 0)),
        out_shape=jax.ShapeDtypeStruct((B, Q, OUT_F), d_A.dtype),
    )(d_A, mask, W, b2)


def kernel(d_A, masked_indices_list, W, b):
    B, Q, _ = d_A.shape
    M = masked_indices_list.shape[1]
    idx = masked_indices_list.astype(jnp.int32)
    mask = _build_mask_sc(idx, B, Q, M)
    return _masked_linear_tc(d_A, mask, W, b.reshape(1, OUT_F), B, Q)


# speedup vs baseline: 1.2595x; 1.0050x over previous
import functools

import jax
import jax.numpy as jnp
from jax import lax
from jax.experimental import pallas as pl
from jax.experimental.pallas import tpu as pltpu
from jax.experimental.pallas import tpu_sc as plsc

IN_F = 512
OUT_F = 32
LANES = 16


def _build_mask_sc(idx, B, Q, M):
    half = Q // 2
    mesh = plsc.VectorSubcoreMesh(core_axis_name="c", subcore_axis_name="s")

    @functools.partial(
        pl.kernel,
        mesh=mesh,
        out_type=jax.ShapeDtypeStruct((B, Q), jnp.float32),
        scratch_types=[
            pltpu.VMEM((M,), jnp.int32),
            pltpu.VMEM((half,), jnp.float32),
        ],
        compiler_params=pltpu.CompilerParams(needs_layout_passes=False),
    )
    def mask_kernel(idx_hbm, mask_hbm, idx_v, mask_v):
        num_c = lax.axis_size("c")
        wid = lax.axis_index("s") * num_c + lax.axis_index("c")
        batch = wid // 2
        base = (wid % 2) * half

        pltpu.sync_copy(idx_hbm.at[batch], idx_v)

        zeros = jnp.zeros((LANES,), jnp.float32)

        def zero_body(i, carry):
            mask_v[pl.ds(i * LANES, LANES)] = zeros
            return carry

        lax.fori_loop(0, half // LANES, zero_body, 0, unroll=8)

        ones = jnp.ones((LANES,), jnp.float32)

        def scat_body(i, carry):
            ids = idx_v[pl.ds(i * LANES, LANES)]
            local = ids - base
            ok = (local >= 0) & (local < half)
            safe = jnp.where(ok, local, 0)
            plsc.store_scatter(mask_v, [safe], ones, mask=ok)
            return carry

        lax.fori_loop(0, M // LANES, scat_body, 0, unroll=8)

        pltpu.sync_copy(mask_v, mask_hbm.at[batch, pl.ds(base, half)])

    return mask_kernel(idx)


def _masked_linear_tc(d_A, mask, W, b2, B, Q):
    grid = (B,)

    def body(x_ref, m_ref, w_ref, b_ref, o_ref):
        acc = lax.dot_general(
            x_ref[0],
            w_ref[...],
            (((1,), (1,)), ((), ())),
            preferred_element_type=jnp.float32,
        )
        m_row = m_ref[pl.ds(pl.program_id(0), 1), :]
        ones_row = jnp.ones((1, OUT_F), jnp.float32)
        mcol = lax.dot_general(
            m_row,
            ones_row,
            (((0,), (0,)), ((), ())),
            preferred_element_type=jnp.float32,
        )
        o_ref[0] = (acc + b_ref[...]) * mcol

    return pl.pallas_call(
        body,
        grid=grid,
        in_specs=[
            pl.BlockSpec((1, Q, IN_F), lambda b: (b, 0, 0)),
            pl.BlockSpec((B, Q), lambda b: (0, 0)),
            pl.BlockSpec((OUT_F, IN_F), lambda b: (0, 0)),
            pl.BlockSpec((1, OUT_F), lambda b: (0, 0)),
        ],
        out_specs=pl.BlockSpec((1, Q, OUT_F), lambda b: (b, 0, 0)),
        out_shape=jax.ShapeDtypeStruct((B, Q, OUT_F), d_A.dtype),
    )(d_A, mask, W, b2)


def kernel(d_A, masked_indices_list, W, b):
    B, Q, _ = d_A.shape
    M = masked_indices_list.shape[1]
    idx = masked_indices_list.astype(jnp.int32)
    mask = _build_mask_sc(idx, B, Q, M)
    return _masked_linear_tc(d_A, mask, W, b.reshape(1, OUT_F), B, Q)
